# baseline (device time: 41365 ns/iter reference)
import jax
import jax.numpy as jnp
from jax import lax
from jax.experimental import pallas as pl
from jax.experimental.pallas import tpu as pltpu

N_DEV = 4
HQ_LOC = 4
HQ = 16
DH = 64
SQ_LOC = 256
QB = 4
B = 2
D_MODEL = 512
ROWS = B * SQ_LOC
HID_LOC = HQ_LOC * DH
PAY_ROWS = 2 * D_MODEL
HALF = D_MODEL


def kernel(x, Wq, K_ext, V_ext, Wo):
    x2 = x.reshape(ROWS, D_MODEL)
    payload = jnp.concatenate(
        [(Wq * 0.125).astype(jnp.bfloat16), Wo.T.astype(jnp.bfloat16)],
        axis=0,
    )

    def body(x_ref, pay_ref, k_hbm, v_hbm, out_ref,
             comm_ref, ctx_ref, xbf_ref, kal_ref, val_ref,
             copy_sem, send_sems, recv_sems):
        my = lax.axis_index("i")
        left = lax.rem(my + (N_DEV - 1), N_DEV)
        right = lax.rem(my + 1, N_DEV)

        copies = []
        for h in range(HQ):
            for src, dst in ((k_hbm, kal_ref), (v_hbm, val_ref)):
                c = pltpu.make_async_copy(
                    src.at[:, :, h, :], dst.at[h], copy_sem
                )
                c.start()
                copies.append(c)

        barrier = pltpu.get_barrier_semaphore()
        for nbr in (left, right):
            pl.semaphore_signal(
                barrier, inc=1,
                device_id=(nbr,), device_id_type=pl.DeviceIdType.MESH,
            )
        pl.semaphore_wait(barrier, 2)

        send_c_right = pltpu.make_async_remote_copy(
            src_ref=pay_ref, dst_ref=comm_ref.at[0],
            send_sem=send_sems.at[0], recv_sem=recv_sems.at[0],
            device_id=(right,), device_id_type=pl.DeviceIdType.MESH,
        )
        send_c_left = pltpu.make_async_remote_copy(
            src_ref=pay_ref, dst_ref=comm_ref.at[1],
            send_sem=send_sems.at[1], recv_sem=recv_sems.at[1],
            device_id=(left,), device_id_type=pl.DeviceIdType.MESH,
        )
        fwd_right = pltpu.make_async_remote_copy(
            src_ref=comm_ref.at[0, pl.ds(0, HALF)],
            dst_ref=comm_ref.at[2, pl.ds(0, HALF)],
            send_sem=send_sems.at[2], recv_sem=recv_sems.at[2],
            device_id=(right,), device_id_type=pl.DeviceIdType.MESH,
        )
        fwd_left = pltpu.make_async_remote_copy(
            src_ref=comm_ref.at[1, pl.ds(HALF, HALF)],
            dst_ref=comm_ref.at[2, pl.ds(HALF, HALF)],
            send_sem=send_sems.at[3], recv_sem=recv_sems.at[3],
            device_id=(left,), device_id_type=pl.DeviceIdType.MESH,
        )

        send_c_right.start()
        send_c_left.start()

        xbf_ref[...] = x_ref[...].astype(jnp.bfloat16)
        for c in copies:
            c.wait()

        def compute(wq_c, woT_c, origin, first):
            q_bf = jnp.dot(
                xbf_ref[...], wq_c, preferred_element_type=jnp.float32
            ).astype(jnp.bfloat16)
            for p in range(QB):
                for t in range(HQ_LOC):
                    head = origin * HQ_LOC + t
                    for b in range(B):
                        r0 = b * SQ_LOC + p * DH
                        q = q_bf[r0:r0 + DH, t * DH:(t + 1) * DH]
                        k = kal_ref[head, b, p * DH:(p + 1) * DH, :]
                        s = lax.dot_general(
                            q, k.astype(jnp.bfloat16),
                            (((1,), (1,)), ((), ())),
                            preferred_element_type=jnp.float32,
                        )
                        e = jnp.exp(s)
                        denom = jnp.sum(e, axis=1, keepdims=True)
                        v = val_ref[head, b, p * DH:(p + 1) * DH, :]
                        ctx = jnp.dot(
                            e.astype(jnp.bfloat16), v.astype(jnp.bfloat16),
                            preferred_element_type=jnp.float32,
                        ) * (1.0 / denom)
                        ctx_ref[r0:r0 + DH, t * DH:(t + 1) * DH] = (
                            ctx.astype(jnp.bfloat16)
                        )
            contrib = lax.dot_general(
                ctx_ref[...], woT_c, (((1,), (1,)), ((), ())),
                preferred_element_type=jnp.float32,
            )
            if first:
                out_ref[...] = contrib
            else:
                out_ref[...] += contrib

        compute(pay_ref[0:HALF, :], pay_ref[HALF:PAY_ROWS, :], my, True)

        send_c_right.wait_recv()
        fwd_right.start()
        compute(comm_ref[0, 0:HALF, :], comm_ref[0, HALF:PAY_ROWS, :],
                lax.rem(my + (N_DEV - 1), N_DEV), False)
        fwd_right.wait_send()

        send_c_left.wait_recv()
        fwd_left.start()
        compute(comm_ref[1, 0:HALF, :], comm_ref[1, HALF:PAY_ROWS, :],
                lax.rem(my + 1, N_DEV), False)

        fwd_right.wait_recv()
        fwd_left.wait_recv()
        compute(comm_ref[2, 0:HALF, :], comm_ref[2, HALF:PAY_ROWS, :],
                lax.rem(my + 2, N_DEV), False)

        send_c_right.wait_send()
        send_c_left.wait_send()
        fwd_left.wait_send()

    out = pl.pallas_call(
        body,
        out_shape=jax.ShapeDtypeStruct((ROWS, D_MODEL), jnp.float32),
        in_specs=[
            pl.BlockSpec(memory_space=pltpu.VMEM),
            pl.BlockSpec(memory_space=pltpu.VMEM),
            pl.BlockSpec(memory_space=pltpu.MemorySpace.HBM),
            pl.BlockSpec(memory_space=pltpu.MemorySpace.HBM),
        ],
        out_specs=pl.BlockSpec(memory_space=pltpu.VMEM),
        scratch_shapes=[
            pltpu.VMEM((3, PAY_ROWS, HID_LOC), jnp.bfloat16),
            pltpu.VMEM((ROWS, HID_LOC), jnp.bfloat16),
            pltpu.VMEM((ROWS, D_MODEL), jnp.bfloat16),
            pltpu.VMEM((HQ, B, SQ_LOC, DH), jnp.float32),
            pltpu.VMEM((HQ, B, SQ_LOC, DH), jnp.float32),
            pltpu.SemaphoreType.DMA,
            pltpu.SemaphoreType.DMA((4,)),
            pltpu.SemaphoreType.DMA((4,)),
        ],
        compiler_params=pltpu.CompilerParams(collective_id=0),
    )(x2, payload, K_ext, V_ext)

    return out.reshape(B, SQ_LOC, D_MODEL)


# device time: 37335 ns/iter; 1.1079x vs baseline; 1.1079x over previous
import jax
import jax.numpy as jnp
from jax import lax
from jax.experimental import pallas as pl
from jax.experimental.pallas import tpu as pltpu

N_DEV = 4
HQ_LOC = 4
HQ = 16
DH = 64
SQ_LOC = 256
QB = 4
B = 2
D_MODEL = 512
ROWS = B * SQ_LOC
RB = B * DH
HID_LOC = HQ_LOC * DH
PAY_ROWS = 2 * D_MODEL
HALF = D_MODEL


def kernel(x, Wq, K_ext, V_ext, Wo):
    x2 = x.reshape(ROWS, D_MODEL)
    k4 = K_ext.reshape(B, QB, DH, HQ, DH)
    v4 = V_ext.reshape(B, QB, DH, HQ, DH)
    payload = jnp.concatenate(
        [(Wq * 0.125).astype(jnp.bfloat16), Wo.T.astype(jnp.bfloat16)],
        axis=0,
    )

    def body(x_ref, pay_ref, k_hbm, v_hbm, out_ref,
             comm_ref, ctx_ref, xbf_ref, acc_ref,
             kal_ref, val_ref, kbf_ref, vbf_ref,
             copy_sem, send_sems, recv_sems):
        my = lax.axis_index("i")
        left = lax.rem(my + (N_DEV - 1), N_DEV)
        right = lax.rem(my + 1, N_DEV)

        copies = []
        for h in range(HQ):
            for b in range(B):
                for src, dst in ((k_hbm, kal_ref), (v_hbm, val_ref)):
                    c = pltpu.make_async_copy(
                        src.at[b, :, :, h, :],
                        dst.at[h, :, pl.ds(b * DH, DH), :],
                        copy_sem,
                    )
                    c.start()
                    copies.append(c)

        barrier = pltpu.get_barrier_semaphore()
        for nbr in (left, right):
            pl.semaphore_signal(
                barrier, inc=1,
                device_id=(nbr,), device_id_type=pl.DeviceIdType.MESH,
            )
        pl.semaphore_wait(barrier, 2)

        send_c_right = pltpu.make_async_remote_copy(
            src_ref=pay_ref, dst_ref=comm_ref.at[0],
            send_sem=send_sems.at[0], recv_sem=recv_sems.at[0],
            device_id=(right,), device_id_type=pl.DeviceIdType.MESH,
        )
        send_c_left = pltpu.make_async_remote_copy(
            src_ref=pay_ref, dst_ref=comm_ref.at[1],
            send_sem=send_sems.at[1], recv_sem=recv_sems.at[1],
            device_id=(left,), device_id_type=pl.DeviceIdType.MESH,
        )
        fwd_right = pltpu.make_async_remote_copy(
            src_ref=comm_ref.at[0, pl.ds(0, HALF)],
            dst_ref=comm_ref.at[2, pl.ds(0, HALF)],
            send_sem=send_sems.at[2], recv_sem=recv_sems.at[2],
            device_id=(right,), device_id_type=pl.DeviceIdType.MESH,
        )
        fwd_left = pltpu.make_async_remote_copy(
            src_ref=comm_ref.at[1, pl.ds(HALF, HALF)],
            dst_ref=comm_ref.at[2, pl.ds(HALF, HALF)],
            send_sem=send_sems.at[3], recv_sem=recv_sems.at[3],
            device_id=(left,), device_id_type=pl.DeviceIdType.MESH,
        )

        send_c_right.start()
        send_c_left.start()

        for p in range(QB):
            for b in range(B):
                xbf_ref[p * RB + b * DH:p * RB + (b + 1) * DH, :] = (
                    x_ref[b * SQ_LOC + p * DH:b * SQ_LOC + (p + 1) * DH, :]
                    .astype(jnp.bfloat16)
                )

        for c in copies:
            c.wait()
        kbf_ref[...] = kal_ref[...].astype(jnp.bfloat16)
        vbf_ref[...] = val_ref[...].astype(jnp.bfloat16)

        row = lax.broadcasted_iota(jnp.int32, (RB, RB), 0)
        col = lax.broadcasted_iota(jnp.int32, (RB, RB), 1)
        maskf = ((row // DH) == (col // DH)).astype(jnp.float32)

        def compute(wq_c, woT_c, origin, first):
            q_bf = jnp.dot(
                xbf_ref[...], wq_c, preferred_element_type=jnp.float32
            ).astype(jnp.bfloat16)
            for p in range(QB):
                for t in range(HQ_LOC):
                    head = origin * HQ_LOC + t
                    q = q_bf[p * RB:(p + 1) * RB, t * DH:(t + 1) * DH]
                    k = kbf_ref[head, p]
                    s = lax.dot_general(
                        q, k, (((1,), (1,)), ((), ())),
                        preferred_element_type=jnp.float32,
                    )
                    e = jnp.exp(s) * maskf
                    denom = jnp.sum(e, axis=1, keepdims=True)
                    ctx = jnp.dot(
                        e.astype(jnp.bfloat16), vbf_ref[head, p],
                        preferred_element_type=jnp.float32,
                    ) * (1.0 / denom)
                    ctx_ref[p * RB:(p + 1) * RB, t * DH:(t + 1) * DH] = (
                        ctx.astype(jnp.bfloat16)
                    )
            contrib = lax.dot_general(
                ctx_ref[...], woT_c, (((1,), (1,)), ((), ())),
                preferred_element_type=jnp.float32,
            )
            if first:
                acc_ref[...] = contrib
            else:
                acc_ref[...] += contrib

        compute(pay_ref[0:HALF, :], pay_ref[HALF:PAY_ROWS, :], my, True)

        send_c_right.wait_recv()
        fwd_right.start()
        compute(comm_ref[0, 0:HALF, :], comm_ref[0, HALF:PAY_ROWS, :],
                lax.rem(my + (N_DEV - 1), N_DEV), False)
        fwd_right.wait_send()

        send_c_left.wait_recv()
        fwd_left.start()
        compute(comm_ref[1, 0:HALF, :], comm_ref[1, HALF:PAY_ROWS, :],
                lax.rem(my + 1, N_DEV), False)

        fwd_right.wait_recv()
        fwd_left.wait_recv()
        compute(comm_ref[2, 0:HALF, :], comm_ref[2, HALF:PAY_ROWS, :],
                lax.rem(my + 2, N_DEV), False)

        for p in range(QB):
            for b in range(B):
                out_ref[b * SQ_LOC + p * DH:b * SQ_LOC + (p + 1) * DH, :] = (
                    acc_ref[p * RB + b * DH:p * RB + (b + 1) * DH, :]
                )

        send_c_right.wait_send()
        send_c_left.wait_send()
        fwd_left.wait_send()

    out = pl.pallas_call(
        body,
        out_shape=jax.ShapeDtypeStruct((ROWS, D_MODEL), jnp.float32),
        in_specs=[
            pl.BlockSpec(memory_space=pltpu.VMEM),
            pl.BlockSpec(memory_space=pltpu.VMEM),
            pl.BlockSpec(memory_space=pltpu.MemorySpace.HBM),
            pl.BlockSpec(memory_space=pltpu.MemorySpace.HBM),
        ],
        out_specs=pl.BlockSpec(memory_space=pltpu.VMEM),
        scratch_shapes=[
            pltpu.VMEM((3, PAY_ROWS, HID_LOC), jnp.bfloat16),
            pltpu.VMEM((ROWS, HID_LOC), jnp.bfloat16),
            pltpu.VMEM((ROWS, D_MODEL), jnp.bfloat16),
            pltpu.VMEM((ROWS, D_MODEL), jnp.float32),
            pltpu.VMEM((HQ, QB, RB, DH), jnp.float32),
            pltpu.VMEM((HQ, QB, RB, DH), jnp.float32),
            pltpu.VMEM((HQ, QB, RB, DH), jnp.bfloat16),
            pltpu.VMEM((HQ, QB, RB, DH), jnp.bfloat16),
            pltpu.SemaphoreType.DMA,
            pltpu.SemaphoreType.DMA((4,)),
            pltpu.SemaphoreType.DMA((4,)),
        ],
        compiler_params=pltpu.CompilerParams(collective_id=0),
    )(x2, payload, k4, v4)

    return out.reshape(B, SQ_LOC, D_MODEL)


# device time: 37266 ns/iter; 1.1100x vs baseline; 1.0019x over previous
import jax
import jax.numpy as jnp
from jax import lax
from jax.experimental import pallas as pl
from jax.experimental.pallas import tpu as pltpu

N_DEV = 4
HQ_LOC = 4
HQ = 16
DH = 64
SQ_LOC = 256
B = 2
GP = 2
TILE = 2 * DH
D_MODEL = 512
ROWS = B * SQ_LOC
HID_LOC = HQ_LOC * DH
PAY_ROWS = 2 * D_MODEL
HALF = D_MODEL


def kernel(x, Wq, K_ext, V_ext, Wo):
    x2 = x.reshape(ROWS, D_MODEL)
    payload = jnp.concatenate(
        [(Wq * 0.125).astype(jnp.bfloat16), Wo.T.astype(jnp.bfloat16)],
        axis=0,
    )

    def body(x_ref, pay_ref, k_hbm, v_hbm, out_ref,
             comm_ref, ctx_ref, xbf_ref, kal_ref, val_ref,
             kbf_ref, vbf_ref, copy_sem, send_sems, recv_sems):
        my = lax.axis_index("i")
        left = lax.rem(my + (N_DEV - 1), N_DEV)
        right = lax.rem(my + 1, N_DEV)

        copies = []
        for h in range(HQ):
            for src, dst in ((k_hbm, kal_ref), (v_hbm, val_ref)):
                c = pltpu.make_async_copy(
                    src.at[:, :, h, :], dst.at[h], copy_sem
                )
                c.start()
                copies.append(c)

        barrier = pltpu.get_barrier_semaphore()
        for nbr in (left, right):
            pl.semaphore_signal(
                barrier, inc=1,
                device_id=(nbr,), device_id_type=pl.DeviceIdType.MESH,
            )
        pl.semaphore_wait(barrier, 2)

        send_c_right = pltpu.make_async_remote_copy(
            src_ref=pay_ref, dst_ref=comm_ref.at[0],
            send_sem=send_sems.at[0], recv_sem=recv_sems.at[0],
            device_id=(right,), device_id_type=pl.DeviceIdType.MESH,
        )
        send_c_left = pltpu.make_async_remote_copy(
            src_ref=pay_ref, dst_ref=comm_ref.at[1],
            send_sem=send_sems.at[1], recv_sem=recv_sems.at[1],
            device_id=(left,), device_id_type=pl.DeviceIdType.MESH,
        )
        fwd_right = pltpu.make_async_remote_copy(
            src_ref=comm_ref.at[0, pl.ds(0, HALF)],
            dst_ref=comm_ref.at[2, pl.ds(0, HALF)],
            send_sem=send_sems.at[2], recv_sem=recv_sems.at[2],
            device_id=(right,), device_id_type=pl.DeviceIdType.MESH,
        )
        fwd_left = pltpu.make_async_remote_copy(
            src_ref=comm_ref.at[1, pl.ds(HALF, HALF)],
            dst_ref=comm_ref.at[2, pl.ds(HALF, HALF)],
            send_sem=send_sems.at[3], recv_sem=recv_sems.at[3],
            device_id=(left,), device_id_type=pl.DeviceIdType.MESH,
        )

        send_c_right.start()
        send_c_left.start()

        xbf_ref[...] = x_ref[...].astype(jnp.bfloat16)

        for c in copies:
            c.wait()
        kbf_ref[...] = kal_ref[...].astype(jnp.bfloat16)
        vbf_ref[...] = val_ref[...].astype(jnp.bfloat16)

        row = lax.broadcasted_iota(jnp.int32, (TILE, TILE), 0)
        col = lax.broadcasted_iota(jnp.int32, (TILE, TILE), 1)
        maskf = ((row // DH) == (col // DH)).astype(jnp.float32)

        def compute(wq_c, woT_c, origin, first):
            q_bf = jnp.dot(
                xbf_ref[...], wq_c, preferred_element_type=jnp.float32
            ).astype(jnp.bfloat16)
            for b in range(B):
                for g in range(GP):
                    r0 = b * SQ_LOC + g * TILE
                    for t in range(HQ_LOC):
                        head = origin * HQ_LOC + t
                        q = q_bf[r0:r0 + TILE, t * DH:(t + 1) * DH]
                        k = kbf_ref[head, b, g * TILE:(g + 1) * TILE, :]
                        s = lax.dot_general(
                            q, k, (((1,), (1,)), ((), ())),
                            preferred_element_type=jnp.float32,
                        )
                        e = jnp.exp(s) * maskf
                        denom = jnp.sum(e, axis=1, keepdims=True)
                        v = vbf_ref[head, b, g * TILE:(g + 1) * TILE, :]
                        ctx = jnp.dot(
                            e.astype(jnp.bfloat16), v,
                            preferred_element_type=jnp.float32,
                        ) * (1.0 / denom)
                        ctx_ref[r0:r0 + TILE, t * DH:(t + 1) * DH] = (
                            ctx.astype(jnp.bfloat16)
                        )
            contrib = lax.dot_general(
                ctx_ref[...], woT_c, (((1,), (1,)), ((), ())),
                preferred_element_type=jnp.float32,
            )
            if first:
                out_ref[...] = contrib
            else:
                out_ref[...] += contrib

        compute(pay_ref[0:HALF, :], pay_ref[HALF:PAY_ROWS, :], my, True)

        send_c_right.wait_recv()
        fwd_right.start()
        compute(comm_ref[0, 0:HALF, :], comm_ref[0, HALF:PAY_ROWS, :],
                lax.rem(my + (N_DEV - 1), N_DEV), False)
        fwd_right.wait_send()

        send_c_left.wait_recv()
        fwd_left.start()
        compute(comm_ref[1, 0:HALF, :], comm_ref[1, HALF:PAY_ROWS, :],
                lax.rem(my + 1, N_DEV), False)

        fwd_right.wait_recv()
        fwd_left.wait_recv()
        compute(comm_ref[2, 0:HALF, :], comm_ref[2, HALF:PAY_ROWS, :],
                lax.rem(my + 2, N_DEV), False)

        send_c_right.wait_send()
        send_c_left.wait_send()
        fwd_left.wait_send()

    out = pl.pallas_call(
        body,
        out_shape=jax.ShapeDtypeStruct((ROWS, D_MODEL), jnp.float32),
        in_specs=[
            pl.BlockSpec(memory_space=pltpu.VMEM),
            pl.BlockSpec(memory_space=pltpu.VMEM),
            pl.BlockSpec(memory_space=pltpu.MemorySpace.HBM),
            pl.BlockSpec(memory_space=pltpu.MemorySpace.HBM),
        ],
        out_specs=pl.BlockSpec(memory_space=pltpu.VMEM),
        scratch_shapes=[
            pltpu.VMEM((3, PAY_ROWS, HID_LOC), jnp.bfloat16),
            pltpu.VMEM((ROWS, HID_LOC), jnp.bfloat16),
            pltpu.VMEM((ROWS, D_MODEL), jnp.bfloat16),
            pltpu.VMEM((HQ, B, SQ_LOC, DH), jnp.float32),
            pltpu.VMEM((HQ, B, SQ_LOC, DH), jnp.float32),
            pltpu.VMEM((HQ, B, SQ_LOC, DH), jnp.bfloat16),
            pltpu.VMEM((HQ, B, SQ_LOC, DH), jnp.bfloat16),
            pltpu.SemaphoreType.DMA,
            pltpu.SemaphoreType.DMA((4,)),
            pltpu.SemaphoreType.DMA((4,)),
        ],
        compiler_params=pltpu.CompilerParams(collective_id=0),
    )(x2, payload, K_ext, V_ext)

    return out.reshape(B, SQ_LOC, D_MODEL)


# device time: 27729 ns/iter; 1.4918x vs baseline; 1.3439x over previous
import jax
import jax.numpy as jnp
from jax import lax
from jax.experimental import pallas as pl
from jax.experimental.pallas import tpu as pltpu

N_DEV = 4
HQ_LOC = 4
HQ = 16
DH = 64
SQ_LOC = 256
B = 2
GP = 2
TILE = 2 * DH
D_MODEL = 512
ROWS = B * SQ_LOC
HID_LOC = HQ_LOC * DH
PAY_ROWS = 2 * D_MODEL
HALF = D_MODEL


def kernel(x, Wq, K_ext, V_ext, Wo):
    x2 = x.reshape(ROWS, D_MODEL)
    kb = jnp.transpose(K_ext.astype(jnp.bfloat16), (2, 0, 1, 3))
    vb = jnp.transpose(V_ext.astype(jnp.bfloat16), (2, 0, 1, 3))
    payload = jnp.concatenate(
        [(Wq * 0.125).astype(jnp.bfloat16), Wo.T.astype(jnp.bfloat16)],
        axis=0,
    )

    def body(x_ref, pay_ref, kbf_ref, vbf_ref, out_ref,
             comm_ref, ctx_ref, xbf_ref, send_sems, recv_sems):
        my = lax.axis_index("i")
        left = lax.rem(my + (N_DEV - 1), N_DEV)
        right = lax.rem(my + 1, N_DEV)

        barrier = pltpu.get_barrier_semaphore()
        for nbr in (left, right):
            pl.semaphore_signal(
                barrier, inc=1,
                device_id=(nbr,), device_id_type=pl.DeviceIdType.MESH,
            )
        pl.semaphore_wait(barrier, 2)

        send_c_right = pltpu.make_async_remote_copy(
            src_ref=pay_ref, dst_ref=comm_ref.at[0],
            send_sem=send_sems.at[0], recv_sem=recv_sems.at[0],
            device_id=(right,), device_id_type=pl.DeviceIdType.MESH,
        )
        send_c_left = pltpu.make_async_remote_copy(
            src_ref=pay_ref, dst_ref=comm_ref.at[1],
            send_sem=send_sems.at[1], recv_sem=recv_sems.at[1],
            device_id=(left,), device_id_type=pl.DeviceIdType.MESH,
        )
        fwd_right = pltpu.make_async_remote_copy(
            src_ref=comm_ref.at[0, pl.ds(0, HALF)],
            dst_ref=comm_ref.at[2, pl.ds(0, HALF)],
            send_sem=send_sems.at[2], recv_sem=recv_sems.at[2],
            device_id=(right,), device_id_type=pl.DeviceIdType.MESH,
        )
        fwd_left = pltpu.make_async_remote_copy(
            src_ref=comm_ref.at[1, pl.ds(HALF, HALF)],
            dst_ref=comm_ref.at[2, pl.ds(HALF, HALF)],
            send_sem=send_sems.at[3], recv_sem=recv_sems.at[3],
            device_id=(left,), device_id_type=pl.DeviceIdType.MESH,
        )

        send_c_right.start()
        send_c_left.start()

        xbf_ref[...] = x_ref[...].astype(jnp.bfloat16)

        row = lax.broadcasted_iota(jnp.int32, (TILE, TILE), 0)
        col = lax.broadcasted_iota(jnp.int32, (TILE, TILE), 1)
        maskf = ((row // DH) == (col // DH)).astype(jnp.float32)

        def compute(wq_c, woT_c, origin, first):
            q_bf = jnp.dot(
                xbf_ref[...], wq_c, preferred_element_type=jnp.float32
            ).astype(jnp.bfloat16)
            for b in range(B):
                for g in range(GP):
                    r0 = b * SQ_LOC + g * TILE
                    for t in range(HQ_LOC):
                        head = origin * HQ_LOC + t
                        q = q_bf[r0:r0 + TILE, t * DH:(t + 1) * DH]
                        k = kbf_ref[head, b, g * TILE:(g + 1) * TILE, :]
                        s = lax.dot_general(
                            q, k, (((1,), (1,)), ((), ())),
                            preferred_element_type=jnp.float32,
                        )
                        e = jnp.exp(s) * maskf
                        denom = jnp.sum(e, axis=1, keepdims=True)
                        v = vbf_ref[head, b, g * TILE:(g + 1) * TILE, :]
                        ctx = jnp.dot(
                            e.astype(jnp.bfloat16), v,
                            preferred_element_type=jnp.float32,
                        ) * (1.0 / denom)
                        ctx_ref[r0:r0 + TILE, t * DH:(t + 1) * DH] = (
                            ctx.astype(jnp.bfloat16)
                        )
            contrib = lax.dot_general(
                ctx_ref[...], woT_c, (((1,), (1,)), ((), ())),
                preferred_element_type=jnp.float32,
            )
            if first:
                out_ref[...] = contrib
            else:
                out_ref[...] += contrib

        compute(pay_ref[0:HALF, :], pay_ref[HALF:PAY_ROWS, :], my, True)

        send_c_right.wait_recv()
        fwd_right.start()
        compute(comm_ref[0, 0:HALF, :], comm_ref[0, HALF:PAY_ROWS, :],
                lax.rem(my + (N_DEV - 1), N_DEV), False)
        fwd_right.wait_send()

        send_c_left.wait_recv()
        fwd_left.start()
        compute(comm_ref[1, 0:HALF, :], comm_ref[1, HALF:PAY_ROWS, :],
                lax.rem(my + 1, N_DEV), False)

        fwd_right.wait_recv()
        fwd_left.wait_recv()
        compute(comm_ref[2, 0:HALF, :], comm_ref[2, HALF:PAY_ROWS, :],
                lax.rem(my + 2, N_DEV), False)

        send_c_right.wait_send()
        send_c_left.wait_send()
        fwd_left.wait_send()

    out = pl.pallas_call(
        body,
        out_shape=jax.ShapeDtypeStruct((ROWS, D_MODEL), jnp.float32),
        in_specs=[
            pl.BlockSpec(memory_space=pltpu.VMEM),
            pl.BlockSpec(memory_space=pltpu.VMEM),
            pl.BlockSpec(memory_space=pltpu.VMEM),
            pl.BlockSpec(memory_space=pltpu.VMEM),
        ],
        out_specs=pl.BlockSpec(memory_space=pltpu.VMEM),
        scratch_shapes=[
            pltpu.VMEM((3, PAY_ROWS, HID_LOC), jnp.bfloat16),
            pltpu.VMEM((ROWS, HID_LOC), jnp.bfloat16),
            pltpu.VMEM((ROWS, D_MODEL), jnp.bfloat16),
            pltpu.SemaphoreType.DMA((4,)),
            pltpu.SemaphoreType.DMA((4,)),
        ],
        compiler_params=pltpu.CompilerParams(collective_id=0),
    )(x2, payload, kb, vb)

    return out.reshape(B, SQ_LOC, D_MODEL)


# device time: 26588 ns/iter; 1.5558x vs baseline; 1.0429x over previous
import jax
import jax.numpy as jnp
from jax import lax
from jax.experimental import pallas as pl
from jax.experimental.pallas import tpu as pltpu

N_DEV = 4
HQ_LOC = 4
HQ = 16
DH = 64
SQ_LOC = 256
B = 2
GP = 2
TILE = 2 * DH
D_MODEL = 512
ROWS = B * SQ_LOC
HID_LOC = HQ_LOC * DH
PAY_ROWS = 2 * D_MODEL
HALF = D_MODEL


def kernel(x, Wq, K_ext, V_ext, Wo):
    x2 = x.reshape(ROWS, D_MODEL)
    kb = jnp.transpose(K_ext.astype(jnp.bfloat16), (2, 0, 1, 3))
    vb = jnp.transpose(V_ext.astype(jnp.bfloat16), (2, 0, 1, 3))
    payload = jnp.concatenate(
        [(Wq * 0.125).astype(jnp.bfloat16), Wo.T.astype(jnp.bfloat16)],
        axis=0,
    )

    def body(x_ref, pay_ref, kbf_ref, vbf_ref, out_ref,
             comm_ref, ctx_ref, xbf_ref, send_sems, recv_sems):
        my = lax.axis_index("i")
        left = lax.rem(my + (N_DEV - 1), N_DEV)
        right = lax.rem(my + 1, N_DEV)

        barrier = pltpu.get_barrier_semaphore()
        for nbr in (left, right):
            pl.semaphore_signal(
                barrier, inc=1,
                device_id=(nbr,), device_id_type=pl.DeviceIdType.MESH,
            )
        pl.semaphore_wait(barrier, 2)

        def half_send(lo, dst_slot, sem_i, dev):
            return pltpu.make_async_remote_copy(
                src_ref=pay_ref.at[pl.ds(lo, HALF)],
                dst_ref=comm_ref.at[dst_slot, pl.ds(lo, HALF)],
                send_sem=send_sems.at[sem_i], recv_sem=recv_sems.at[sem_i],
                device_id=(dev,), device_id_type=pl.DeviceIdType.MESH,
            )

        send_a_right = half_send(0, 0, 0, right)
        send_a_left = half_send(0, 1, 1, left)
        send_b_right = half_send(HALF, 0, 2, right)
        send_b_left = half_send(HALF, 1, 3, left)
        fwd_a_right = pltpu.make_async_remote_copy(
            src_ref=comm_ref.at[0, pl.ds(0, HALF)],
            dst_ref=comm_ref.at[2, pl.ds(0, HALF)],
            send_sem=send_sems.at[4], recv_sem=recv_sems.at[4],
            device_id=(right,), device_id_type=pl.DeviceIdType.MESH,
        )
        fwd_b_left = pltpu.make_async_remote_copy(
            src_ref=comm_ref.at[1, pl.ds(HALF, HALF)],
            dst_ref=comm_ref.at[2, pl.ds(HALF, HALF)],
            send_sem=send_sems.at[5], recv_sem=recv_sems.at[5],
            device_id=(left,), device_id_type=pl.DeviceIdType.MESH,
        )

        send_a_right.start()
        send_a_left.start()
        send_b_right.start()
        send_b_left.start()

        xbf_ref[...] = x_ref[...].astype(jnp.bfloat16)

        row = lax.broadcasted_iota(jnp.int32, (TILE, TILE), 0)
        col = lax.broadcasted_iota(jnp.int32, (TILE, TILE), 1)
        maskf = ((row // DH) == (col // DH)).astype(jnp.float32)

        def attn(wq_c, origin):
            q_bf = jnp.dot(
                xbf_ref[...], wq_c, preferred_element_type=jnp.float32
            ).astype(jnp.bfloat16)
            for b in range(B):
                for g in range(GP):
                    r0 = b * SQ_LOC + g * TILE
                    for t in range(HQ_LOC):
                        head = origin * HQ_LOC + t
                        q = q_bf[r0:r0 + TILE, t * DH:(t + 1) * DH]
                        k = kbf_ref[head, b, g * TILE:(g + 1) * TILE, :]
                        s = lax.dot_general(
                            q, k, (((1,), (1,)), ((), ())),
                            preferred_element_type=jnp.float32,
                        )
                        e = jnp.exp(s) * maskf
                        denom = jnp.sum(e, axis=1, keepdims=True)
                        v = vbf_ref[head, b, g * TILE:(g + 1) * TILE, :]
                        ctx = jnp.dot(
                            e.astype(jnp.bfloat16), v,
                            preferred_element_type=jnp.float32,
                        ) * (1.0 / denom)
                        ctx_ref[r0:r0 + TILE, t * DH:(t + 1) * DH] = (
                            ctx.astype(jnp.bfloat16)
                        )

        def proj(woT_c, first):
            contrib = lax.dot_general(
                ctx_ref[...], woT_c, (((1,), (1,)), ((), ())),
                preferred_element_type=jnp.float32,
            )
            if first:
                out_ref[...] = contrib
            else:
                out_ref[...] += contrib

        attn(pay_ref[0:HALF, :], my)
        proj(pay_ref[HALF:PAY_ROWS, :], True)

        send_a_right.wait_recv()
        fwd_a_right.start()
        attn(comm_ref[0, 0:HALF, :], lax.rem(my + (N_DEV - 1), N_DEV))
        send_b_right.wait_recv()
        proj(comm_ref[0, HALF:PAY_ROWS, :], False)
        fwd_a_right.wait_send()

        send_a_left.wait_recv()
        attn(comm_ref[1, 0:HALF, :], lax.rem(my + 1, N_DEV))
        send_b_left.wait_recv()
        fwd_b_left.start()
        proj(comm_ref[1, HALF:PAY_ROWS, :], False)

        fwd_a_right.wait_recv()
        attn(comm_ref[2, 0:HALF, :], lax.rem(my + 2, N_DEV))
        fwd_b_left.wait_recv()
        proj(comm_ref[2, HALF:PAY_ROWS, :], False)

        send_a_right.wait_send()
        send_a_left.wait_send()
        send_b_right.wait_send()
        send_b_left.wait_send()
        fwd_b_left.wait_send()

    out = pl.pallas_call(
        body,
        out_shape=jax.ShapeDtypeStruct((ROWS, D_MODEL), jnp.float32),
        in_specs=[
            pl.BlockSpec(memory_space=pltpu.VMEM),
            pl.BlockSpec(memory_space=pltpu.VMEM),
            pl.BlockSpec(memory_space=pltpu.VMEM),
            pl.BlockSpec(memory_space=pltpu.VMEM),
        ],
        out_specs=pl.BlockSpec(memory_space=pltpu.VMEM),
        scratch_shapes=[
            pltpu.VMEM((3, PAY_ROWS, HID_LOC), jnp.bfloat16),
            pltpu.VMEM((ROWS, HID_LOC), jnp.bfloat16),
            pltpu.VMEM((ROWS, D_MODEL), jnp.bfloat16),
            pltpu.SemaphoreType.DMA((6,)),
            pltpu.SemaphoreType.DMA((6,)),
        ],
        compiler_params=pltpu.CompilerParams(collective_id=0),
    )(x2, payload, kb, vb)

    return out.reshape(B, SQ_LOC, D_MODEL)
